# baseline (speedup 1.0000x reference)
import functools

import jax
import jax.numpy as jnp
from jax import lax
from jax.experimental import pallas as pl
from jax.experimental.pallas import tpu as pltpu
from jax.experimental.pallas import tpu_sc as plsc

NUM_HEADS = 16
NEG = float(jnp.finfo(jnp.float32).min)
L = 16


@functools.lru_cache(maxsize=None)
def _make_sc_kernel(n_tok, d_model, n_ehs, s_enc, n_posb):
    info = plsc.get_sparse_core_info()
    nc, ns = info.num_cores, info.num_subcores
    nw = nc * ns
    bpw = n_tok // nw
    epw = n_ehs // nw
    nchunk = 4
    eh = epw // nchunk
    mpw = n_tok // nw
    mg = 4
    ngroup = mpw // mg
    zpw = n_posb // nw
    s_dec = n_tok
    cpr = s_dec // L
    mesh = plsc.VectorSubcoreMesh(core_axis_name="c", subcore_axis_name="s")

    @functools.partial(
        pl.kernel,
        mesh=mesh,
        cost_estimate=pl.CostEstimate(
            flops=16 * n_tok * d_model,
            transcendentals=0,
            bytes_accessed=64 * 1024 * 1024,
        ),
        out_type=(
            jax.ShapeDtypeStruct((n_tok, d_model), jnp.float32),
            jax.ShapeDtypeStruct((n_ehs, d_model), jnp.float32),
            jax.ShapeDtypeStruct((s_dec, s_dec), jnp.float32),
            jax.ShapeDtypeStruct((s_enc,), jnp.float32),
            jax.ShapeDtypeStruct((n_posb,), jnp.float32),
        ),
        scratch_types=[
            pltpu.VMEM((bpw,), jnp.int32),
            pltpu.VMEM((bpw, d_model), jnp.float32),
            pltpu.VMEM((2, eh, d_model), jnp.float32),
            pltpu.VMEM((2, mg, s_dec), jnp.float32),
            pltpu.VMEM((s_dec,), jnp.float32),
            pltpu.VMEM((s_enc,), jnp.float32),
            pltpu.VMEM((zpw,), jnp.float32),
            pltpu.SemaphoreType.DMA,
            pltpu.SemaphoreType.DMA,
            pltpu.SemaphoreType.DMA((2,)),
            pltpu.SemaphoreType.DMA((2,)),
            pltpu.SemaphoreType.DMA((2,)),
        ],
    )
    def sc_k(table_hbm, idx_hbm, ehs_hbm, dmask_hbm, emask_hbm,
             hid_out, ehs_out, dmask_out, emask_out, posb_out,
             idx_v, rows_v, ebuf, mbuf, bbuf, encb, zbuf,
             sem_g, sem_go, sem_ei, sem_eo, sem_mo):
        wid = lax.axis_index("s") * nc + lax.axis_index("c")
        base = wid * bpw
        ebase = wid * epw
        mbase = wid * mpw

        def cin(ci, slot):
            return pltpu.async_copy(
                ehs_hbm.at[pl.ds(ebase + ci * eh, eh)], ebuf.at[slot],
                sem_ei.at[slot])

        def cout(ci, slot):
            return pltpu.async_copy(
                ebuf.at[slot], ehs_out.at[pl.ds(ebase + ci * eh, eh)],
                sem_eo.at[slot])

        ein = [None] * nchunk
        eout = [None] * nchunk
        ein[0] = cin(0, 0)
        ein[1] = cin(1, 1)
        pltpu.sync_copy(idx_hbm.at[pl.ds(base, bpw)], idx_v)
        g = pltpu.async_copy(table_hbm.at[idx_v], rows_v, sem_g)
        for ci in range(nchunk):
            slot = ci % 2
            ein[ci].wait()
            eout[ci] = cout(ci, slot)
            if ci + 2 < nchunk:
                eout[ci].wait()
                ein[ci + 2] = cin(ci + 2, slot)
        g.wait()
        go = pltpu.async_copy(rows_v, hid_out.at[pl.ds(base, bpw)], sem_go)

        pltpu.sync_copy(dmask_hbm, bbuf)
        neg = jnp.full((L,), NEG, dtype=jnp.float32)
        one = jnp.full((L,), 1.0, dtype=jnp.float32)

        def bfix(c, _):
            mC = bbuf[pl.ds(c * L, L)]
            bbuf[pl.ds(c * L, L)] = (one - mC) * neg
            return 0

        lax.fori_loop(0, cpr, bfix, 0)

        ramp = lax.broadcasted_iota(jnp.int32, (L,), 0)
        mo = [None, None]
        for gi in range(ngroup):
            slot = gi % 2
            if mo[slot] is not None:
                mo[slot].wait()
            r0 = mbase + gi * mg

            def mrow(c, _, _r0=r0, _slot=slot):
                col = ramp + c * L
                bC = bbuf[pl.ds(c * L, L)]
                for i in range(mg):
                    v = jnp.where(col <= _r0 + i, bC, neg)
                    mbuf[_slot, i, pl.ds(c * L, L)] = v
                return 0

            lax.fori_loop(0, cpr, mrow, 0)
            mo[slot] = pltpu.async_copy(
                mbuf.at[slot], dmask_out.at[pl.ds(r0, mg)], sem_mo.at[slot])

        zero = jnp.zeros((L,), dtype=jnp.float32)

        def zfill(c, _):
            zbuf[pl.ds(c * L, L)] = zero
            return 0

        lax.fori_loop(0, zpw // L, zfill, 0)
        pltpu.sync_copy(zbuf, posb_out.at[pl.ds(wid * zpw, zpw)])

        @pl.when(wid == 0)
        def _():
            pltpu.sync_copy(emask_hbm, encb)

            def efix(c, _):
                mC = encb[pl.ds(c * L, L)]
                encb[pl.ds(c * L, L)] = (one - mC) * neg
                return 0

            lax.fori_loop(0, s_enc // L, efix, 0)
            pltpu.sync_copy(encb, emask_out)

        mo[0].wait()
        mo[1].wait()
        go.wait()

    return sc_k


def kernel(encoder_hidden_states, encoder_position_bias, decoder_input_ids,
           decoder_attention_mask, encoder_attention_mask, embedding_weight):
    b, s_dec = decoder_input_ids.shape
    vocab, d_model = embedding_weight.shape
    _, s_enc, _ = encoder_hidden_states.shape
    ids_flat = decoder_input_ids.reshape(-1)
    ehs_flat = encoder_hidden_states.reshape(b * s_enc, d_model)
    n_posb = NUM_HEADS * b * s_dec

    sc_k = _make_sc_kernel(b * s_dec, d_model, b * s_enc, s_enc, n_posb)
    hid, ehs_out, dmask, emask, posb = sc_k(
        embedding_weight, ids_flat, ehs_flat,
        decoder_attention_mask.reshape(-1), encoder_attention_mask.reshape(-1))

    decoder_hidden_states = hid.reshape(b, s_dec, d_model)
    ehs_out = ehs_out.reshape(encoder_hidden_states.shape)
    dec_ext = dmask.reshape(b, 1, s_dec, s_dec)
    enc_ext = emask.reshape(b, 1, 1, s_enc)
    decoder_position_bias = posb.reshape(b, NUM_HEADS, s_dec, 1)

    return (ehs_out, encoder_position_bias, decoder_hidden_states,
            enc_ext, dec_ext, decoder_position_bias)


# speedup vs baseline: 1.0611x; 1.0611x over previous
import functools

import jax
import jax.numpy as jnp
from jax import lax
from jax.experimental import pallas as pl
from jax.experimental.pallas import tpu as pltpu
from jax.experimental.pallas import tpu_sc as plsc

NUM_HEADS = 16
NEG = float(jnp.finfo(jnp.float32).min)


def _mask_body(dec_mask_ref, enc_mask_ref, dec_out_ref, enc_out_ref):
    i = pl.program_id(0)
    _, _, R, S = dec_out_ref.shape
    row = i * R + lax.broadcasted_iota(jnp.int32, (1, 1, R, S), 2)
    col = lax.broadcasted_iota(jnp.int32, (1, 1, R, S), 3)
    causal = jnp.where(col <= row, 1.0, 0.0)
    m = dec_mask_ref[0, :].astype(jnp.float32)[None, None, None, :]
    dec_out_ref[...] = (1.0 - causal * m) * NEG
    e = enc_mask_ref[0, :].astype(jnp.float32)[None, None, None, :]
    enc_out_ref[...] = (1.0 - e) * NEG


def _make_masks(dec_mask, enc_mask):
    _, s_dec = dec_mask.shape
    _, s_enc = enc_mask.shape
    rows_per_step = 512
    grid = s_dec // rows_per_step
    return pl.pallas_call(
        _mask_body,
        grid=(grid,),
        in_specs=[
            pl.BlockSpec((1, s_dec), lambda i: (0, 0)),
            pl.BlockSpec((1, s_enc), lambda i: (0, 0)),
        ],
        out_specs=[
            pl.BlockSpec((1, 1, rows_per_step, s_dec), lambda i: (0, 0, i, 0)),
            pl.BlockSpec((1, 1, 1, s_enc), lambda i: (0, 0, 0, 0)),
        ],
        out_shape=[
            jax.ShapeDtypeStruct((1, 1, s_dec, s_dec), jnp.float32),
            jax.ShapeDtypeStruct((1, 1, 1, s_enc), jnp.float32),
        ],
    )(dec_mask, enc_mask)


@functools.lru_cache(maxsize=None)
def _make_sc_gather(n_tok, d_model, n_ehs):
    info = plsc.get_sparse_core_info()
    nc, ns = info.num_cores, info.num_subcores
    nw = nc * ns
    bpw = n_tok // nw
    gh = bpw // 2
    epw = n_ehs // nw
    nchunk = 8
    nslot = 4
    eh = epw // nchunk
    mesh = plsc.VectorSubcoreMesh(core_axis_name="c", subcore_axis_name="s")

    @functools.partial(
        pl.kernel,
        mesh=mesh,
        out_type=(
            jax.ShapeDtypeStruct((n_tok, d_model), jnp.float32),
            jax.ShapeDtypeStruct((n_ehs, d_model), jnp.float32),
        ),
        scratch_types=[
            pltpu.VMEM((bpw,), jnp.int32),
            pltpu.VMEM((2, gh, d_model), jnp.float32),
            pltpu.VMEM((nslot, eh, d_model), jnp.float32),
            pltpu.SemaphoreType.DMA((2,)),
            pltpu.SemaphoreType.DMA((2,)),
            pltpu.SemaphoreType.DMA((nslot,)),
            pltpu.SemaphoreType.DMA((nslot,)),
        ],
    )
    def gather_k(table_hbm, idx_hbm, ehs_hbm, hid_out, ehs_out,
                 idx_v, rows_v, ebuf, sem_g, sem_go, sem_ei, sem_eo):
        wid = lax.axis_index("s") * nc + lax.axis_index("c")
        base = wid * bpw
        ebase = wid * epw

        def cin(ci, slot):
            return pltpu.async_copy(
                ehs_hbm.at[pl.ds(ebase + ci * eh, eh)], ebuf.at[slot],
                sem_ei.at[slot])

        def cout(ci, slot):
            return pltpu.async_copy(
                ebuf.at[slot], ehs_out.at[pl.ds(ebase + ci * eh, eh)],
                sem_eo.at[slot])

        ein = [None] * nchunk
        eout = [None] * nchunk
        for ci in range(nslot):
            ein[ci] = cin(ci, ci)
        pltpu.sync_copy(idx_hbm.at[pl.ds(base, bpw)], idx_v)
        g0 = pltpu.async_copy(
            table_hbm.at[idx_v.at[pl.ds(0, gh)]], rows_v.at[0], sem_g.at[0])
        g1 = pltpu.async_copy(
            table_hbm.at[idx_v.at[pl.ds(gh, gh)]], rows_v.at[1], sem_g.at[1])
        for ci in range(nchunk):
            slot = ci % nslot
            ein[ci].wait()
            eout[ci] = cout(ci, slot)
            if ci + nslot < nchunk:
                eout[ci].wait()
                ein[ci + nslot] = cin(ci + nslot, slot)
        g0.wait()
        go0 = pltpu.async_copy(
            rows_v.at[0], hid_out.at[pl.ds(base, gh)], sem_go.at[0])
        g1.wait()
        go1 = pltpu.async_copy(
            rows_v.at[1], hid_out.at[pl.ds(base + gh, gh)], sem_go.at[1])
        for ci in range(nchunk - nslot, nchunk):
            eout[ci].wait()
        go0.wait()
        go1.wait()

    return gather_k


def kernel(encoder_hidden_states, encoder_position_bias, decoder_input_ids,
           decoder_attention_mask, encoder_attention_mask, embedding_weight):
    b, s_dec = decoder_input_ids.shape
    vocab, d_model = embedding_weight.shape
    _, s_enc, _ = encoder_hidden_states.shape
    ids_flat = decoder_input_ids.reshape(-1)
    ehs_flat = encoder_hidden_states.reshape(b * s_enc, d_model)

    gather_k = _make_sc_gather(b * s_dec, d_model, b * s_enc)
    hid, ehs_out = gather_k(embedding_weight, ids_flat, ehs_flat)
    decoder_hidden_states = hid.reshape(b, s_dec, d_model)
    ehs_out = ehs_out.reshape(encoder_hidden_states.shape)

    dec_ext, enc_ext = _make_masks(decoder_attention_mask, encoder_attention_mask)

    decoder_position_bias = jnp.zeros((b, NUM_HEADS, s_dec, 1), dtype=jnp.float32)

    return (ehs_out, encoder_position_bias, decoder_hidden_states,
            enc_ext, dec_ext, decoder_position_bias)
